# Initial kernel scaffold; baseline (speedup 1.0000x reference)
import jax
import jax.numpy as jnp
from jax.experimental import pallas as pl


def kernel(x, edge_index, energy, candidates, u, W_fc0, b_fc0, ln0_g, ln0_b, ln1_g, ln1_b, ln2_g, ln2_b, comb_w, W_fc1, b_fc1):
    raise NotImplementedError("write your pallas kernel here")


# speedup vs baseline: 1.0003x; 1.0003x over previous
import functools

import jax
import jax.numpy as jnp
from jax.experimental import pallas as pl

N_NODES = 10000
N_EDGES = 320000
N_CAND = 100000
D_FEAT = 128
HID = 64
N_CLASS = 10
LAYERS = 2
SAMPLING_RATE = 0.1
TAU = 0.3
EPS = 1e-8

ROW_BLK = 2000


def _proj_ln_relu_body(x_ref, w_ref, b_ref, g_ref, bb_ref, o_ref):
    h = jnp.dot(x_ref[...], w_ref[...], preferred_element_type=jnp.float32)
    h = h + b_ref[...][None, :]
    mu = jnp.mean(h, axis=-1, keepdims=True)
    var = jnp.var(h, axis=-1, keepdims=True)
    h = (h - mu) / jnp.sqrt(var + 1e-5) * g_ref[...][None, :] + bb_ref[...][None, :]
    o_ref[...] = jnp.maximum(h, 0.0)


def _proj_ln_relu(x, W, b, g, bb):
    grid = (N_NODES // ROW_BLK,)
    return pl.pallas_call(
        _proj_ln_relu_body,
        grid=grid,
        in_specs=[
            pl.BlockSpec((ROW_BLK, D_FEAT), lambda i: (i, 0)),
            pl.BlockSpec((D_FEAT, HID), lambda i: (0, 0)),
            pl.BlockSpec((HID,), lambda i: (0,)),
            pl.BlockSpec((HID,), lambda i: (0,)),
            pl.BlockSpec((HID,), lambda i: (0,)),
        ],
        out_specs=pl.BlockSpec((ROW_BLK, HID), lambda i: (i, 0)),
        out_shape=jax.ShapeDtypeStruct((N_NODES, HID), jnp.float32),
    )(x, W, b, g, bb)


def _combine_out_body(h0_ref, h1_ref, h2_ref, cw_ref, w_ref, b_ref, o_ref):
    cw = cw_ref[...]
    hsum = h0_ref[...] * cw[0] + h1_ref[...] * cw[1] + h2_ref[...] * cw[2]
    o_ref[...] = jnp.dot(hsum, w_ref[...], preferred_element_type=jnp.float32) + b_ref[...][None, :]


def _combine_out(h0, h1, h2, comb_w, W, b):
    grid = (N_NODES // ROW_BLK,)
    return pl.pallas_call(
        _combine_out_body,
        grid=grid,
        in_specs=[
            pl.BlockSpec((ROW_BLK, HID), lambda i: (i, 0)),
            pl.BlockSpec((ROW_BLK, HID), lambda i: (i, 0)),
            pl.BlockSpec((ROW_BLK, HID), lambda i: (i, 0)),
            pl.BlockSpec((LAYERS + 1,), lambda i: (0,)),
            pl.BlockSpec((HID, N_CLASS), lambda i: (0, 0)),
            pl.BlockSpec((N_CLASS,), lambda i: (0,)),
        ],
        out_specs=pl.BlockSpec((ROW_BLK, N_CLASS), lambda i: (i, 0)),
        out_shape=jax.ShapeDtypeStruct((N_NODES, N_CLASS), jnp.float32),
    )(h0, h1, h2, comb_w, W, b)


def _layer_norm(h, g, b):
    mu = jnp.mean(h, axis=-1, keepdims=True)
    var = jnp.var(h, axis=-1, keepdims=True)
    return (h - mu) / jnp.sqrt(var + 1e-5) * g + b


def _mask_edges(vals, T, D, M, N):
    E = T.shape[0]
    sorted_idx = jnp.argsort(-T)
    sorted_T = T[sorted_idx]
    mask_high = (D >= jnp.mean(D)) & (M >= jnp.mean(M)) & (T >= jnp.mean(T))
    mask = mask_high[sorted_idx]
    cum = jnp.cumsum(mask.astype(jnp.float32))
    mu_k = cum / (jnp.sum(mask_high.astype(jnp.float32)) + EPS)
    keep = E - N
    non_self_T = sorted_T[:keep]
    mu_k = mu_k[:keep]
    T_smooth = (jnp.concatenate([non_self_T[:1], non_self_T[:-1]]) + non_self_T + jnp.concatenate([non_self_T[1:], non_self_T[-1:]])) / 3.0
    T_ratios = T_smooth[:-1] / (T_smooth[1:] + EPS)
    T_ratios = jnp.concatenate([T_ratios, jnp.ones((1,), dtype=T_smooth.dtype)])
    TGap = mu_k * T_ratios
    L = jnp.argmax(TGap)
    rank = jnp.zeros((E,), dtype=jnp.int32).at[sorted_idx].set(jnp.arange(E, dtype=jnp.int32))
    retain = (rank >= L).astype(vals.dtype)
    return vals * retain


def _add_candidates(xd, energy, candidates, u):
    ni = candidates[0]
    nj = candidates[1]
    D_add = jnp.linalg.norm(xd[ni] - xd[nj], axis=1)
    M_add = energy[ni] * energy[nj]
    T_add = D_add * M_add
    sorted_idx = jnp.argsort(-T_add)
    si = ni[sorted_idx]
    sj = nj[sorted_idx]
    sT = T_add[sorted_idx]
    nT = (sT - jnp.min(sT)) / (jnp.max(sT) - jnp.min(sT) + EPS)
    p = 1.0 - nT
    scale = SAMPLING_RATE / (jnp.mean(p) + EPS)
    p = jnp.minimum(p * scale, 1.0)
    logits = jnp.stack([jnp.log(1.0 - p + EPS), jnp.log(p + EPS)], axis=-1)
    g = -jnp.log(-jnp.log(u + EPS) + EPS)
    y = jax.nn.softmax((logits + g) / TAU, axis=-1)
    soft = y[:, 1]
    return si, sj, soft


def _sym_norm(rows, cols, vals, N, order, layers):
    loop = jnp.arange(N, dtype=rows.dtype)
    rows2 = jnp.concatenate([rows, loop])
    cols2 = jnp.concatenate([cols, loop])
    vals2 = jnp.concatenate([vals, jnp.ones((N,), dtype=vals.dtype)])
    deg = jax.ops.segment_sum(vals2, rows2, num_segments=N) + 1e-8
    dis = deg ** -0.5
    nv = dis[rows2] * dis[cols2] * ((layers - order + 1) / layers)
    nv = jnp.nan_to_num(nv, nan=0.0, posinf=0.0, neginf=0.0)
    return rows2, cols2, nv


def _spmm(rows, cols, vals, h, N):
    msgs = vals[:, None] * h[rows]
    return jax.ops.segment_sum(msgs, cols, num_segments=N)


def kernel(x, edge_index, energy, candidates, u, W_fc0, b_fc0, ln0_g, ln0_b,
           ln1_g, ln1_b, ln2_g, ln2_b, comb_w, W_fc1, b_fc1):
    N = x.shape[0]
    h = x @ W_fc0 + b_fc0
    h = _layer_norm(h, ln0_g, ln0_b)
    h = jnp.maximum(h, 0.0)
    outs = [h]
    er = edge_index[0]
    ec = edge_index[1]
    ln_gs = [ln1_g, ln2_g]
    ln_bs = [ln1_b, ln2_b]
    for i in range(LAYERS):
        xd = h
        D = jnp.linalg.norm(xd[er] - xd[ec], axis=1)
        M = energy[er] * energy[ec]
        T = D * M
        vals = jnp.ones((er.shape[0],), dtype=jnp.float32)
        vals = _mask_edges(vals, T, D, M, N)
        si, sj, soft = _add_candidates(xd, energy, candidates, u)
        rows = jnp.concatenate([er, si])
        cols = jnp.concatenate([ec, sj])
        allv = jnp.concatenate([vals, soft])
        rows, cols, nv = _sym_norm(rows, cols, allv, N, i + 1, LAYERS)
        h = _spmm(rows, cols, nv, h, N)
        h = _layer_norm(h, ln_gs[i], ln_bs[i])
        h = jnp.maximum(h, 0.0)
        outs.append(h)
    return _combine_out(outs[0], outs[1], outs[2], comb_w, W_fc1, b_fc1)


# speedup vs baseline: 1.1055x; 1.1052x over previous
import dataclasses
import functools

import jax
import jax.numpy as jnp
from jax import lax
from jax.experimental import pallas as pl
from jax.experimental.pallas import tpu as pltpu
from jax.experimental.pallas import tpu_sc as plsc

N_NODES = 10000
N_EDGES = 320000
N_CAND = 100000
D_FEAT = 128
HID = 64
N_CLASS = 10
LAYERS = 2
SAMPLING_RATE = 0.1
TAU = 0.3
EPS = 1e-8

NL = 16
NT = 16
NSUB = NT * NL

E_CHUNK = 1251
E_PAD = NSUB * E_CHUNK
E_REG = NL * E_CHUNK
E_ROWS = (E_REG + 127) // 128
E_REGP = E_ROWS * 128

C_CHUNK = 401
C_PAD = NSUB * C_CHUNK
C_REG = NL * C_CHUNK
C_ROWS = (C_REG + 127) // 128
C_REGP = C_ROWS * 128

TRASH = 128
IBLK = 128
E_NBLK = (E_CHUNK + IBLK - 1) // IBLK
C_NBLK = (C_CHUNK + IBLK - 1) // IBLK
RADIX = 256


def _vgather(x, idx):
    return x.at[idx].get(mode="promise_in_bounds")


def _sort_pass(srcb, psrc, kbuf, pbuf, hist, nxt, loff, gt_v, bt_v,
               carry_ref, stage_k, stage_p, stage_pos, scratch4k,
               gt_sh, bt_sh, dstk_sh, dstp_sh, sem,
               tile, chunk, reg, nblk, ushift, with_payload, dst_cap):
    li = lax.iota(jnp.int32, NL)
    lidx = li * chunk
    lastlane = jnp.full((NL,), NL - 1, jnp.int32)
    base = tile * reg

    pltpu.sync_copy(srcb.at[pl.ds(base, reg)], kbuf.at[pl.ds(0, reg)])
    if with_payload:
        pltpu.sync_copy(psrc.at[pl.ds(base, reg)], pbuf.at[pl.ds(0, reg)])

    @pl.loop(0, RADIX * NL, step=NL)
    def _(j):
        hist[pl.ds(j, NL)] = jnp.zeros((NL,), jnp.int32)

    @pl.loop(0, chunk)
    def _(i):
        k16 = plsc.load_gather(kbuf, [lidx + i])
        d = lax.shift_right_logical(k16, ushift) & jnp.int32(255)
        a = d * NL + li
        c = plsc.load_gather(hist, [a])
        plsc.store_scatter(hist, [a], c + 1)

    @pl.loop(0, RADIX)
    def _(d):
        row = hist[pl.ds(d * NL, NL)]
        cum = plsc.cumsum(row)
        loff[pl.ds(d * NL, NL)] = cum - row
        tot = _vgather(cum, lastlane)
        nxt[pl.ds(d * NL, NL)] = tot

    @pl.loop(0, RADIX, step=NL)
    def _(dv):
        acc = jnp.zeros((NL,), jnp.int32)
        for j in range(NL):
            tj = nxt[pl.ds((dv + j) * NL, NL)]
            acc = jnp.where(li == j, tj, acc)
        gt_v[pl.ds(dv, NL)] = acc

    pltpu.sync_copy(gt_v, gt_sh.at[pl.ds(tile * RADIX, RADIX)])
    plsc.subcore_barrier()

    @pl.when(tile == 0)
    def _():
        pltpu.sync_copy(gt_sh, scratch4k)
        carry_ref[pl.ds(0, NL)] = jnp.zeros((NL,), jnp.int32)

        @pl.loop(0, RADIX)
        def _(d):
            vd = plsc.load_gather(scratch4k, [li * RADIX + d])
            cum = plsc.cumsum(vd)
            excl = cum - vd
            tot = _vgather(cum, lastlane)
            carry = carry_ref[pl.ds(0, NL)]
            bt_v[pl.ds(d * NL, NL)] = carry + excl
            carry_ref[pl.ds(0, NL)] = carry + tot

        pltpu.sync_copy(bt_v, bt_sh)

    plsc.subcore_barrier()

    pltpu.sync_copy(bt_sh, bt_v)
    tsplat = jnp.zeros((NL,), jnp.int32) + tile

    @pl.loop(0, RADIX)
    def _(d):
        brow = bt_v[pl.ds(d * NL, NL)]
        mine = _vgather(brow, tsplat)
        nxt[pl.ds(d * NL, NL)] = mine + loff[pl.ds(d * NL, NL)]

    @pl.loop(0, nblk)
    def _(b):
        i0 = b * IBLK
        i1 = jnp.minimum(i0 + IBLK, chunk)

        @pl.loop(0, IBLK)
        def _(io):
            i = i0 + io

            @pl.when(i < i1)
            def _():
                k16 = plsc.load_gather(kbuf, [lidx + i])
                d = lax.shift_right_logical(k16, ushift) & jnp.int32(255)
                a = d * NL + li
                pos = plsc.load_gather(nxt, [a])
                plsc.store_scatter(nxt, [a], pos + 1)
                j = io * NL
                r = lax.shift_right_logical(j, 7)
                c = j & 127
                stage_k[r, pl.ds(c, NL)] = k16
                stage_pos[r, pl.ds(c, NL)] = pos
                if with_payload:
                    p16 = plsc.load_gather(pbuf, [lidx + i])
                    stage_p[r, pl.ds(c, NL)] = p16

            @pl.when(i >= i1)
            def _():
                j = io * NL
                r = lax.shift_right_logical(j, 7)
                c = j & 127
                stage_pos[r, pl.ds(c, NL)] = jnp.full((NL,), dst_cap, jnp.int32) + li
                stage_k[r, pl.ds(c, NL)] = jnp.zeros((NL,), jnp.int32)
                if with_payload:
                    stage_p[r, pl.ds(c, NL)] = jnp.zeros((NL,), jnp.int32)

        cps = []
        for r in range(NL):
            cps.append(pltpu.make_async_copy(
                stage_k.at[r], dstk_sh.at[stage_pos.at[r]], sem))
            if with_payload:
                cps.append(pltpu.make_async_copy(
                    stage_p.at[r], dstp_sh.at[stage_pos.at[r]], sem))
        for cp in cps:
            cp.start()
        for cp in cps:
            cp.wait()

    plsc.subcore_barrier()


def _radix_sort_kernel(ke_hbm, kc_hbm, eko_hbm, cko_hbm, cio_hbm,
                       kbuf, pbuf, hist, nxt, loff, gt_v, bt_v, carry_ref,
                       stage_k, stage_p, stage_pos, scratch4k,
                       bufA, bufB, pbufA, pbufB, gt_sh, bt_sh,
                       sem):
    core = lax.axis_index("c")
    tile = lax.axis_index("s")
    li = lax.iota(jnp.int32, NL)

    @pl.when(core == 0)
    def _():
        chunk, reg, nblk = E_CHUNK, E_REG, E_NBLK
        base = tile * reg
        pltpu.sync_copy(ke_hbm.at[pl.ds(base, reg)], kbuf.at[pl.ds(0, reg)])
        pltpu.sync_copy(kbuf.at[pl.ds(0, reg)], bufA.at[pl.ds(base, reg)])

        @pl.loop(0, 2)
        def _(it):
            s0 = it * jnp.int32(16)
            _sort_pass(bufA, None, kbuf, pbuf, hist, nxt, loff, gt_v, bt_v,
                       carry_ref, stage_k, stage_p, stage_pos, scratch4k,
                       gt_sh, bt_sh, bufB, None, sem,
                       tile, chunk, reg, nblk, s0, False, E_PAD)
            _sort_pass(bufB, None, kbuf, pbuf, hist, nxt, loff, gt_v, bt_v,
                       carry_ref, stage_k, stage_p, stage_pos, scratch4k,
                       gt_sh, bt_sh, bufA, None, sem,
                       tile, chunk, reg, nblk, s0 + 8, False, E_PAD)

        pltpu.sync_copy(bufA.at[pl.ds(base, reg)], kbuf.at[pl.ds(0, reg)])
        pltpu.sync_copy(kbuf.at[pl.ds(0, reg)], eko_hbm.at[pl.ds(base, reg)])

    @pl.when(core == 1)
    def _():
        chunk, reg, nblk = C_CHUNK, C_REG, C_NBLK
        base = tile * reg
        pltpu.sync_copy(kc_hbm.at[pl.ds(base, reg)], kbuf.at[pl.ds(0, reg)])
        pltpu.sync_copy(kbuf.at[pl.ds(0, reg)], bufA.at[pl.ds(base, reg)])

        @pl.loop(0, reg, step=NL)
        def _(j):
            pbuf[pl.ds(j, NL)] = base + j + li

        pltpu.sync_copy(pbuf.at[pl.ds(0, reg)], pbufA.at[pl.ds(base, reg)])

        @pl.loop(0, 2)
        def _(it):
            s0 = it * jnp.int32(16)
            _sort_pass(bufA, pbufA, kbuf, pbuf, hist, nxt, loff, gt_v, bt_v,
                       carry_ref, stage_k, stage_p, stage_pos, scratch4k,
                       gt_sh, bt_sh, bufB, pbufB, sem,
                       tile, chunk, reg, nblk, s0, True, C_PAD)
            _sort_pass(bufB, pbufB, kbuf, pbuf, hist, nxt, loff, gt_v, bt_v,
                       carry_ref, stage_k, stage_p, stage_pos, scratch4k,
                       gt_sh, bt_sh, bufA, pbufA, sem,
                       tile, chunk, reg, nblk, s0 + 8, True, C_PAD)

        pltpu.sync_copy(bufA.at[pl.ds(base, reg)], kbuf.at[pl.ds(0, reg)])
        pltpu.sync_copy(kbuf.at[pl.ds(0, reg)], cko_hbm.at[pl.ds(base, reg)])
        pltpu.sync_copy(pbufA.at[pl.ds(base, reg)], pbuf.at[pl.ds(0, reg)])
        pltpu.sync_copy(pbuf.at[pl.ds(0, reg)], cio_hbm.at[pl.ds(base, reg)])


def _sc_sorts(T_e, mask_e, T_c):
    mesh = plsc.VectorSubcoreMesh(core_axis_name="c", subcore_axis_name="s",
                                  num_cores=2)
    cp = pltpu.CompilerParams()
    if "needs_layout_passes" in pltpu.CompilerParams.__dataclass_fields__:
        cp = dataclasses.replace(cp, needs_layout_passes=False)
    f = pl.kernel(
        _radix_sort_kernel,
        compiler_params=cp,
        out_type=[
            jax.ShapeDtypeStruct((E_PAD,), jnp.int32),
            jax.ShapeDtypeStruct((C_PAD,), jnp.int32),
            jax.ShapeDtypeStruct((C_PAD,), jnp.int32),
        ],
        mesh=mesh,
        scratch_types=[
            pltpu.VMEM((E_REGP,), jnp.int32),
            pltpu.VMEM((C_REGP,), jnp.int32),
            pltpu.VMEM((RADIX * NL,), jnp.int32),
            pltpu.VMEM((RADIX * NL,), jnp.int32),
            pltpu.VMEM((RADIX * NL,), jnp.int32),
            pltpu.VMEM((RADIX,), jnp.int32),
            pltpu.VMEM((RADIX * NL,), jnp.int32),
            pltpu.VMEM((NL,), jnp.int32),
            pltpu.VMEM((NL, 128), jnp.int32),
            pltpu.VMEM((NL, 128), jnp.int32),
            pltpu.VMEM((NL, 128), jnp.int32),
            pltpu.VMEM((NT * RADIX,), jnp.int32),
            pltpu.VMEM_SHARED((E_PAD + TRASH,), jnp.int32),
            pltpu.VMEM_SHARED((E_PAD + TRASH,), jnp.int32),
            pltpu.VMEM_SHARED((C_PAD + TRASH,), jnp.int32),
            pltpu.VMEM_SHARED((C_PAD + TRASH,), jnp.int32),
            pltpu.VMEM_SHARED((NT * RADIX,), jnp.int32),
            pltpu.VMEM_SHARED((RADIX * NL,), jnp.int32),
            pltpu.SemaphoreType.DMA,
        ],
    )
    ke = _encode_keys(T_e, mask_e)
    ke = jnp.where(jnp.arange(E_PAD) < N_EDGES, ke, jnp.uint32(0xFFFFFFFF))
    kc = _encode_keys(T_c, jnp.zeros((C_PAD,), jnp.int32))
    kc = jnp.where(jnp.arange(C_PAD) < N_CAND, kc, jnp.uint32(0xFFFFFFFF))
    ek, ck, ci = f(lax.bitcast_convert_type(ke, jnp.int32),
                   lax.bitcast_convert_type(kc, jnp.int32))
    return (lax.bitcast_convert_type(ek, jnp.uint32),
            lax.bitcast_convert_type(ck, jnp.uint32), ci)


def _layer_norm(h, g, b):
    mu = jnp.mean(h, axis=-1, keepdims=True)
    var = jnp.var(h, axis=-1, keepdims=True)
    return (h - mu) / jnp.sqrt(var + 1e-5) * g + b


def _decode_keys(skey):
    sk = lax.shift_right_logical(skey, jnp.uint32(1)) | jnp.uint32(0x80000000)
    tbits = ~(sk + jnp.uint32(1))
    return lax.bitcast_convert_type(tbits, jnp.float32)


def _encode_keys(T, mask_bit):
    tbits = lax.bitcast_convert_type(T, jnp.uint32)
    return ((~tbits - jnp.uint32(1)) & jnp.uint32(0x7FFFFFFF)) * jnp.uint32(2) \
        + mask_bit.astype(jnp.uint32)


def _mask_edges_from_sort(T, D, M, mask_high, sorted_keys, N):
    E = T.shape[0]
    sorted_keys = sorted_keys[:E]
    sorted_T = _decode_keys(sorted_keys)
    mask = (sorted_keys & jnp.uint32(1)).astype(jnp.float32)
    cum = jnp.cumsum(mask)
    mu_k = cum / (jnp.sum(mask_high.astype(jnp.float32)) + EPS)
    keep = E - N
    non_self_T = sorted_T[:keep]
    mu_k = mu_k[:keep]
    T_smooth = (jnp.concatenate([non_self_T[:1], non_self_T[:-1]]) + non_self_T
                + jnp.concatenate([non_self_T[1:], non_self_T[-1:]])) / 3.0
    T_ratios = T_smooth[:-1] / (T_smooth[1:] + EPS)
    T_ratios = jnp.concatenate([T_ratios, jnp.ones((1,), T_smooth.dtype)])
    TGap = mu_k * T_ratios
    L = jnp.argmax(TGap)
    s_L = sorted_keys[L]
    skey = _encode_keys(T, mask_high)
    retain = (skey >= s_L).astype(jnp.float32)
    return retain


def kernel(x, edge_index, energy, candidates, u, W_fc0, b_fc0, ln0_g, ln0_b,
           ln1_g, ln1_b, ln2_g, ln2_b, comb_w, W_fc1, b_fc1):
    N = x.shape[0]
    h = x @ W_fc0 + b_fc0
    h = _layer_norm(h, ln0_g, ln0_b)
    h = jax.nn.relu(h)
    outs = [h]
    er = edge_index[0]
    ec = edge_index[1]
    ni = candidates[0]
    nj = candidates[1]
    ln_gs = [ln1_g, ln2_g]
    ln_bs = [ln1_b, ln2_b]
    for i in range(LAYERS):
        xd = h
        D = jnp.linalg.norm(xd[er] - xd[ec], axis=1)
        M = energy[er] * energy[ec]
        T = D * M
        D_add = jnp.linalg.norm(xd[ni] - xd[nj], axis=1)
        M_add = energy[ni] * energy[nj]
        T_add = D_add * M_add

        mask_high = ((D >= jnp.mean(D)) & (M >= jnp.mean(M)) & (T >= jnp.mean(T)))
        T_e = jnp.pad(T, (0, E_PAD - N_EDGES))
        m_e = jnp.pad(mask_high.astype(jnp.int32), (0, E_PAD - N_EDGES))
        T_c = jnp.pad(T_add, (0, C_PAD - N_CAND))
        ek_sorted, ck_sorted, cidx_sorted = _sc_sorts(T_e, m_e, T_c)

        vals = _mask_edges_from_sort(T, D, M, mask_high, ek_sorted, N)

        sidx = cidx_sorted[:N_CAND]
        sT = _decode_keys(ck_sorted[:N_CAND])
        si = ni[sidx]
        sj = nj[sidx]
        nT = (sT - jnp.min(sT)) / (jnp.max(sT) - jnp.min(sT) + EPS)
        p = 1.0 - nT
        scale = SAMPLING_RATE / (jnp.mean(p) + EPS)
        p = jnp.minimum(p * scale, 1.0)
        logits = jnp.stack([jnp.log(1.0 - p + EPS), jnp.log(p + EPS)], axis=-1)
        g = -jnp.log(-jnp.log(u + EPS) + EPS)
        y = jax.nn.softmax((logits + g) / TAU, axis=-1)
        soft = y[:, 1]

        rows = jnp.concatenate([er, si])
        cols = jnp.concatenate([ec, sj])
        allv = jnp.concatenate([vals, soft])
        loop = jnp.arange(N, dtype=rows.dtype)
        rows2 = jnp.concatenate([rows, loop])
        cols2 = jnp.concatenate([cols, loop])
        vals2 = jnp.concatenate([allv, jnp.ones((N,), jnp.float32)])
        deg = jax.ops.segment_sum(vals2, rows2, num_segments=N) + 1e-8
        dis = deg ** -0.5
        nv = dis[rows2] * dis[cols2] * ((LAYERS - (i + 1) + 1) / LAYERS)
        nv = jnp.nan_to_num(nv, nan=0.0, posinf=0.0, neginf=0.0)
        msgs = nv[:, None] * h[rows2]
        h = jax.ops.segment_sum(msgs, cols2, num_segments=N)
        h = _layer_norm(h, ln_gs[i], ln_bs[i])
        h = jax.nn.relu(h)
        outs.append(h)
    hsum = outs[0] * comb_w[0]
    for i in range(1, LAYERS + 1):
        hsum = hsum + outs[i] * comb_w[i]
    return hsum @ W_fc1 + b_fc1
